# Initial kernel scaffold; baseline (speedup 1.0000x reference)
import jax
import jax.numpy as jnp
from jax.experimental import pallas as pl


def kernel(x, edge_index, batch, dropout, edge_attr, device, params):
    raise NotImplementedError("write your pallas kernel here")


# speedup vs baseline: 6.2350x; 6.2350x over previous
import functools

import jax
import jax.numpy as jnp
from jax import lax
from jax.experimental import pallas as pl
from jax.experimental.pallas import tpu as pltpu
from jax.experimental.pallas import tpu_sc as plsc

N = 10000
E = 320000
D = 128
ED = 16
C = 128
NCLASS = 10
NGRAPH = 64
NHID = 3
NLAYER = 1 + NHID

NC, NS, LN = 2, 16, 16
WL = 16
NW = NC * NS
CPW = 320
NPAD = NW * CPW
CH = 128
EPAD = 320512
RB = 1024
NBLK = NPAD // RB

_f32 = jnp.float32
_i32 = jnp.int32

_mesh = plsc.VectorSubcoreMesh(
    core_axis_name="c", subcore_axis_name="s", num_cores=NC, num_subcores=NS)


def _wid():
    return lax.axis_index("c") * NS + lax.axis_index("s")


@functools.partial(
    pl.kernel,
    out_type=jax.ShapeDtypeStruct((NPAD, WL), _f32),
    mesh=_mesh,
    scratch_types=[
        pltpu.VMEM((CPW, WL), _f32),
        pltpu.VMEM((CH,), _i32),
        pltpu.VMEM((CH, WL), _f32),
        pltpu.VMEM((NW, LN), _i32),
    ],
)
def _sc_deg(w_sorted, col_s, ebounds, deg_out, dacc, col_v, w_v, eb_v):
    w = _wid()
    c0 = w * CPW

    def zero_row(i, _):
        dacc[i, :] = jnp.zeros((16,), _f32)
        return _

    lax.fori_loop(0, CPW, zero_row, None)

    pltpu.sync_copy(ebounds, eb_v)
    ebv = eb_v[w, :]
    e_lo = ebv[0]
    e_hi = ebv[1]
    e_start = (e_lo // 8) * 8
    nchunks = (e_hi - e_start + CH - 1) // CH

    def chunk(ci, _):
        e0 = pl.multiple_of(e_start + ci * CH, 8)
        pltpu.sync_copy(col_s.at[pl.ds(e0, CH)], col_v)
        pltpu.sync_copy(w_sorted.at[pl.ds(e0, CH), :], w_v)

        def edge16(kb, _):
            base = kb * LN
            col16 = col_v[pl.ds(base, LN)]
            for k in range(LN):
                ce = col16[k]
                ok = jnp.logical_and(ce >= c0, ce < c0 + CPW)

                @pl.when(ok)
                def _go():
                    plsc.addupdate(dacc.at[ce - c0, :], w_v[base + k, :])

            return _

        lax.fori_loop(0, CH // LN, edge16, None)
        return _

    lax.fori_loop(0, nchunks, chunk, None)
    pltpu.sync_copy(dacc, deg_out.at[pl.ds(c0, CPW), :])


def _make_sc_layer(layer):
    @functools.partial(
        pl.kernel,
        out_type=jax.ShapeDtypeStruct((NPAD, C), _f32),
        mesh=_mesh,
        scratch_types=[
            pltpu.VMEM((CPW, C), _f32),
            pltpu.VMEM((CH,), _i32),
            pltpu.VMEM((CH,), _i32),
            pltpu.VMEM((CH, WL), _f32),
            pltpu.VMEM((CH, C), _f32),
            pltpu.VMEM((NW, LN), _i32),
            pltpu.SemaphoreType.DMA,
        ],
    )
    def _sc_layer(g_hbm, row_s, col_s, w_sorted, ebounds, out, acc, row_v,
                  col_v, w_v, rows_v, eb_v, sem):
        w = _wid()
        c0 = w * CPW

        def zero_row(i, _):
            for j in range(C // LN):
                acc[i, pl.ds(LN * j, LN)] = jnp.zeros((LN,), _f32)
            return _

        lax.fori_loop(0, CPW, zero_row, None)

        pltpu.sync_copy(ebounds, eb_v)
        ebv = eb_v[w, :]
        e_lo = ebv[0]
        e_hi = ebv[1]
        e_start = (e_lo // 8) * 8
        nchunks = (e_hi - e_start + CH - 1) // CH

        def chunk(ci, _):
            e0 = pl.multiple_of(e_start + ci * CH, 8)
            pltpu.sync_copy(row_s.at[pl.ds(e0, CH)], row_v)
            pltpu.sync_copy(col_s.at[pl.ds(e0, CH)], col_v)
            pltpu.sync_copy(w_sorted.at[pl.ds(e0, CH), :], w_v)
            pltpu.async_copy(g_hbm.at[row_v], rows_v, sem).wait()

            def edge16(kb, _):
                base = kb * LN
                col16 = col_v[pl.ds(base, LN)]
                for k in range(LN):
                    ce = col16[k]
                    ok = jnp.logical_and(ce >= c0, ce < c0 + CPW)

                    @pl.when(ok)
                    def _go():
                        loc = ce - c0
                        wrow = w_v[base + k, :]
                        we = jnp.full((LN,), wrow[layer], _f32)
                        for j in range(C // LN):
                            sl = pl.ds(LN * j, LN)
                            plsc.addupdate(acc.at[loc, sl], rows_v[base + k, sl] * we)

                return _

            lax.fori_loop(0, CH // LN, edge16, None)
            return _

        lax.fori_loop(0, nchunks, chunk, None)
        pltpu.sync_copy(acc, out.at[pl.ds(c0, CPW), :])

    return _sc_layer


_sc_layers = [_make_sc_layer(l) for l in range(NLAYER)]


RPW = 320

@functools.partial(
    pl.kernel,
    out_type=jax.ShapeDtypeStruct((NW, NGRAPH, C), _f32),
    mesh=_mesh,
    scratch_types=[
        pltpu.VMEM((NGRAPH, C), _f32),
        pltpu.VMEM((RPW,), _i32),
        pltpu.VMEM((RPW, C), _f32),
        pltpu.SemaphoreType.DMA,
    ],
)
def _sc_segmax(h1, batch, out, acc, b_v, h_v, sem):
    w = _wid()
    r0 = w * RPW
    nrows = jnp.minimum(RPW, jnp.maximum(N - r0, 0))

    def zero_row(i, _):
        for j in range(C // LN):
            acc[i, pl.ds(LN * j, LN)] = jnp.full((LN,), -jnp.inf, _f32)
        return _

    lax.fori_loop(0, NGRAPH, zero_row, None)

    pltpu.sync_copy(batch.at[pl.ds(r0, RPW)], b_v)
    pltpu.sync_copy(h1.at[pl.ds(r0, RPW), :], h_v)

    def row16(kb, _):
        base = kb * LN
        b16 = b_v[pl.ds(base, LN)]
        for k in range(LN):
            g = b16[k]
            ok = base + k < nrows

            @pl.when(ok)
            def _go():
                for j in range(C // LN):
                    sl = pl.ds(LN * j, LN)
                    acc[g, sl] = jnp.maximum(acc[g, sl], h_v[base + k, sl])

        return _

    lax.fori_loop(0, RPW // LN, row16, None)
    pltpu.sync_copy(acc, out.at[w])


def _tc_edgew(ea_ref, w1_ref, b1_ref, w2_ref, b2_ref, o_ref):
    h = jnp.dot(ea_ref[...], w1_ref[...], preferred_element_type=_f32)
    h = jnp.maximum(h + b1_ref[...], 0.0)
    z = jnp.dot(h, w2_ref[...], preferred_element_type=_f32) + b2_ref[...]
    o_ref[...] = jax.nn.sigmoid(z)


def _tc_dinv(deg_ref, o_ref):
    o_ref[...] = lax.rsqrt(deg_ref[...] + 1.0)


def _make_tc_pre(layer, with_bn):
    def body(x_ref, mom_ref, w_ref, dinv_ref, g_ref):
        x = x_ref[...]
        if with_bn:
            mom = mom_ref[...]
            mu = jnp.sum(mom[:, 0, :], axis=0, keepdims=True) / N
            msq = jnp.sum(mom[:, 1, :], axis=0, keepdims=True) / N
            inv = lax.rsqrt(msq - mu * mu + 1e-5)
            x = jnp.maximum((x - mu) * inv, 0.0)
        hw = jnp.dot(x, w_ref[...], preferred_element_type=_f32)
        g_ref[...] = hw * dinv_ref[:, layer:layer + 1]
    return body


def _make_tc_post(layer):
    def body(s_ref, g_ref, dinv_ref, b_ref, o_ref, mom_ref):
        bidx = pl.program_id(0)
        out = dinv_ref[:, layer:layer + 1] * (s_ref[...] + g_ref[...]) + b_ref[...]
        o_ref[...] = out
        rowid = bidx * RB + lax.broadcasted_iota(_i32, (RB, 1), 0)
        m = (rowid < N).astype(_f32)
        om = out * m
        mom_ref[0, 0, :] = jnp.sum(om, axis=0)
        mom_ref[0, 1, :] = jnp.sum(om * out, axis=0)
    return body


def _tc_post3(s_ref, g_ref, dinv_ref, b_ref, h0_ref, o_ref):
    out = dinv_ref[:, 3:4] * (s_ref[...] + g_ref[...]) + b_ref[...]
    o_ref[...] = jnp.maximum(out + h0_ref[...], 0.0)


def _tc_final(pp_ref, w_ref, b_ref, o_ref):
    pooled = jnp.max(pp_ref[...], axis=0)
    o_ref[...] = jnp.dot(pooled, w_ref[...], preferred_element_type=_f32) + b_ref[...]


def _rowspec(width):
    return pl.BlockSpec((RB, width), lambda b: (b, 0))


def _fullspec(shape):
    nd = len(shape)
    return pl.BlockSpec(shape, lambda b: (0,) * nd)


def kernel(x, edge_index, batch, dropout, edge_attr, device, params):
    row, col = edge_index[0], edge_index[1]

    perm = jnp.argsort(col)
    col_s = jnp.sort(col)
    row_s = row[perm]
    eb = jnp.searchsorted(col_s, jnp.arange(NW + 1, dtype=_i32) * CPW).astype(_i32)
    ebounds = jnp.zeros((NW, LN), _i32)
    ebounds = ebounds.at[:, 0].set(eb[:-1]).at[:, 1].set(eb[1:])
    col_sp = jnp.pad(col_s, (0, EPAD - E), constant_values=jnp.int32(2**30))
    row_sp = jnp.pad(row_s, (0, EPAD - E))
    batch_p = jnp.pad(batch, (0, NPAD - N))
    x_p = jnp.pad(x, ((0, NPAD - N), (0, 0)))
    ea_p = jnp.pad(edge_attr[perm], ((0, EPAD - E), (0, 0)))

    convs = [params["conv1"]] + list(params["hidden"])
    w1cat = jnp.concatenate([c["mlp_W1"] for c in convs], axis=1)
    b1cat = jnp.concatenate([c["mlp_b1"] for c in convs])[None, :]
    w2bd = jnp.zeros((NLAYER * ED, WL), _f32)
    b2v = jnp.zeros((1, WL), _f32)
    for l, c in enumerate(convs):
        w2bd = w2bd.at[l * ED:(l + 1) * ED, l].set(c["mlp_W2"][:, 0])
        b2v = b2v.at[0, l].set(c["mlp_b2"][0])

    w_sorted = pl.pallas_call(
        _tc_edgew,
        grid=(EPAD // RB,),
        in_specs=[_rowspec(ED), _fullspec((ED, NLAYER * ED)),
                  _fullspec((1, NLAYER * ED)), _fullspec((NLAYER * ED, WL)),
                  _fullspec((1, WL))],
        out_specs=_rowspec(WL),
        out_shape=jax.ShapeDtypeStruct((EPAD, WL), _f32),
    )(ea_p, w1cat, b1cat, w2bd, b2v)

    deg = _sc_deg(w_sorted, col_sp, ebounds)

    dinv = pl.pallas_call(
        _tc_dinv,
        grid=(NBLK,),
        in_specs=[_rowspec(WL)],
        out_specs=_rowspec(WL),
        out_shape=jax.ShapeDtypeStruct((NPAD, WL), _f32),
    )(deg)

    h = x_p
    mom = None
    h0 = None
    for l in range(NLAYER):
        with_bn = l > 0
        in_arrs = [h, mom if with_bn else jnp.zeros((NBLK, 2, C), _f32),
                   convs[l]["lin_W"], dinv]
        g = pl.pallas_call(
            _make_tc_pre(l, with_bn),
            grid=(NBLK,),
            in_specs=[_rowspec(C), _fullspec((NBLK, 2, C)),
                      _fullspec((C, C)), _rowspec(WL)],
            out_specs=_rowspec(C),
            out_shape=jax.ShapeDtypeStruct((NPAD, C), _f32),
        )(*in_arrs)

        s = _sc_layers[l](g, row_sp, col_sp, w_sorted, ebounds)

        bias = convs[l]["bias"][None, :]
        if l < NLAYER - 1:
            h, mom = pl.pallas_call(
                _make_tc_post(l),
                grid=(NBLK,),
                in_specs=[_rowspec(C), _rowspec(C), _rowspec(WL),
                          _fullspec((1, C))],
                out_specs=[_rowspec(C), pl.BlockSpec((1, 2, C), lambda b: (b, 0, 0))],
                out_shape=[jax.ShapeDtypeStruct((NPAD, C), _f32),
                           jax.ShapeDtypeStruct((NBLK, 2, C), _f32)],
            )(s, g, dinv, bias)
            if l == 0:
                h0 = h
        else:
            h1 = pl.pallas_call(
                _tc_post3,
                grid=(NBLK,),
                in_specs=[_rowspec(C), _rowspec(C), _rowspec(WL),
                          _fullspec((1, C)), _rowspec(C)],
                out_specs=_rowspec(C),
                out_shape=jax.ShapeDtypeStruct((NPAD, C), _f32),
            )(s, g, dinv, bias, h0)

    pooled_part = _sc_segmax(h1, batch_p)

    linw = jnp.pad(params["lin_W"], ((0, 0), (0, C - NCLASS)))
    linb = jnp.pad(params["lin_b"], (0, C - NCLASS))[None, :]
    logits = pl.pallas_call(
        _tc_final,
        grid=(1,),
        in_specs=[_fullspec((NW, NGRAPH, C)), _fullspec((C, C)),
                  _fullspec((1, C))],
        out_specs=_fullspec((NGRAPH, C)),
        out_shape=jax.ShapeDtypeStruct((NGRAPH, C), _f32),
    )(pooled_part, linw, linb)
    return logits[:, :NCLASS]


# speedup vs baseline: 7.2920x; 1.1695x over previous
import functools

import jax
import jax.numpy as jnp
from jax import lax
from jax.experimental import pallas as pl
from jax.experimental.pallas import tpu as pltpu
from jax.experimental.pallas import tpu_sc as plsc

N = 10000
E = 320000
D = 128
ED = 16
C = 128
NCLASS = 10
NGRAPH = 64
NHID = 3
NLAYER = 1 + NHID

NC, NS, LN = 2, 16, 16
WL = 16
NW = NC * NS
CPW = 320
NPAD = NW * CPW
CH = 128
EPAD = 320512
RB = 1024
NBLK = NPAD // RB

_f32 = jnp.float32
_i32 = jnp.int32

_mesh = plsc.VectorSubcoreMesh(
    core_axis_name="c", subcore_axis_name="s", num_cores=NC, num_subcores=NS)


def _wid():
    return lax.axis_index("c") * NS + lax.axis_index("s")


@functools.partial(
    pl.kernel,
    out_type=jax.ShapeDtypeStruct((NPAD, WL), _f32),
    mesh=_mesh,
    scratch_types=[
        pltpu.VMEM((CPW, WL), _f32),
        pltpu.VMEM((CH,), _i32),
        pltpu.VMEM((CH, WL), _f32),
        pltpu.VMEM((NW, LN), _i32),
    ],
)
def _sc_deg(w_sorted, col_s, ebounds, deg_out, dacc, col_v, w_v, eb_v):
    w = _wid()
    c0 = w * CPW

    def zero_row(i, _):
        dacc[i, :] = jnp.zeros((16,), _f32)
        return _

    lax.fori_loop(0, CPW, zero_row, None)

    pltpu.sync_copy(ebounds, eb_v)
    ebv = eb_v[w, :]
    e_lo = ebv[0]
    e_hi = ebv[1]
    e_start = (e_lo // 8) * 8
    nchunks = (e_hi - e_start + CH - 1) // CH

    def chunk(ci, _):
        e0 = pl.multiple_of(e_start + ci * CH, 8)
        pltpu.sync_copy(col_s.at[pl.ds(e0, CH)], col_v)
        pltpu.sync_copy(w_sorted.at[pl.ds(e0, CH), :], w_v)

        def edge16(kb, _):
            base = kb * LN
            col16 = col_v[pl.ds(base, LN)]
            for k in range(LN):
                ce = col16[k]
                ok = jnp.logical_and(ce >= c0, ce < c0 + CPW)

                @pl.when(ok)
                def _go():
                    plsc.addupdate(dacc.at[ce - c0, :], w_v[base + k, :])

            return _

        lax.fori_loop(0, CH // LN, edge16, None)
        return _

    lax.fori_loop(0, nchunks, chunk, None)
    pltpu.sync_copy(dacc, deg_out.at[pl.ds(c0, CPW), :])


@functools.partial(
    pl.kernel,
    out_type=jax.ShapeDtypeStruct((NPAD, C), _f32),
    mesh=_mesh,
    scratch_types=[
        pltpu.VMEM((CPW + 1, C), _f32),
        pltpu.VMEM((2, CH), _i32),
        pltpu.VMEM((2, CH), _i32),
        pltpu.VMEM((2, CH), _f32),
        pltpu.VMEM((2, CH, C), _f32),
        pltpu.VMEM((NW, LN), _i32),
        pltpu.SemaphoreType.DMA,
        pltpu.SemaphoreType.DMA,
    ],
)
def _sc_layer(g_hbm, row_s, col_s, w_l, ebounds, out, acc, row_v,
              col_v, w_v, rows_v, eb_v, sem0, sem1):
    w = _wid()
    c0 = w * CPW
    sems = (sem0, sem1)

    def zero_row(i, _):
        for j in range(C // LN):
            acc[i, pl.ds(LN * j, LN)] = jnp.zeros((LN,), _f32)
        return _

    lax.fori_loop(0, CPW + 1, zero_row, None)

    pltpu.sync_copy(ebounds, eb_v)
    ebv = eb_v[w, :]
    e_lo = ebv[0]
    e_hi = ebv[1]
    e_start = (e_lo // 8) * 8
    nchunks = (e_hi - e_start + CH - 1) // CH

    def issue(ci, slot):
        e0 = pl.multiple_of(e_start + ci * CH, 8)
        pltpu.sync_copy(row_s.at[pl.ds(e0, CH)], row_v.at[slot])
        pltpu.sync_copy(col_s.at[pl.ds(e0, CH)], col_v.at[slot])
        pltpu.sync_copy(w_l.at[pl.ds(e0, CH)], w_v.at[slot])
        pltpu.async_copy(g_hbm.at[row_v.at[slot]], rows_v.at[slot],
                         sems[slot])

    def wait_gather(slot):
        pltpu.make_async_copy(g_hbm.at[row_v.at[slot]], rows_v.at[slot],
                              sems[slot]).wait()

    def process(slot):
        def grp(kb, _):
            base = kb * LN
            loc16 = col_v[slot, pl.ds(base, LN)] - c0
            ok16 = jnp.logical_and(loc16 >= 0, loc16 < CPW)
            loc16 = jnp.where(ok16, loc16, CPW)
            w16 = w_v[slot, pl.ds(base, LN)]
            for k in range(LN):
                loc = loc16[k]
                we = jnp.full((LN,), w16[k], _f32)
                for j in range(C // LN):
                    sl = pl.ds(LN * j, LN)
                    plsc.addupdate(acc.at[loc, sl],
                                   rows_v[slot, base + k, sl] * we)
            return _

        lax.fori_loop(0, CH // LN, grp, None)

    @pl.when(nchunks > 0)
    def _prime():
        issue(0, 0)

    def pair(p, _):
        for b in range(2):
            ci = p * 2 + b

            @pl.when(ci < nchunks)
            def _do(ci=ci, b=b):
                @pl.when(ci + 1 < nchunks)
                def _iss():
                    issue(ci + 1, 1 - b)

                wait_gather(b)
                process(b)

        return _

    lax.fori_loop(0, (nchunks + 1) // 2, pair, None)
    pltpu.sync_copy(acc.at[pl.ds(0, CPW), :], out.at[pl.ds(c0, CPW), :])


RPW = 320

@functools.partial(
    pl.kernel,
    out_type=jax.ShapeDtypeStruct((NW, NGRAPH, C), _f32),
    mesh=_mesh,
    scratch_types=[
        pltpu.VMEM((NGRAPH, C), _f32),
        pltpu.VMEM((RPW,), _i32),
        pltpu.VMEM((RPW, C), _f32),
        pltpu.SemaphoreType.DMA,
    ],
)
def _sc_segmax(h1, batch, out, acc, b_v, h_v, sem):
    w = _wid()
    r0 = w * RPW
    nrows = jnp.minimum(RPW, jnp.maximum(N - r0, 0))

    def zero_row(i, _):
        for j in range(C // LN):
            acc[i, pl.ds(LN * j, LN)] = jnp.full((LN,), -jnp.inf, _f32)
        return _

    lax.fori_loop(0, NGRAPH, zero_row, None)

    pltpu.sync_copy(batch.at[pl.ds(r0, RPW)], b_v)
    pltpu.sync_copy(h1.at[pl.ds(r0, RPW), :], h_v)

    def row16(kb, _):
        base = kb * LN
        b16 = b_v[pl.ds(base, LN)]
        for k in range(LN):
            g = b16[k]
            ok = base + k < nrows

            @pl.when(ok)
            def _go():
                for j in range(C // LN):
                    sl = pl.ds(LN * j, LN)
                    acc[g, sl] = jnp.maximum(acc[g, sl], h_v[base + k, sl])

        return _

    lax.fori_loop(0, RPW // LN, row16, None)
    pltpu.sync_copy(acc, out.at[w])


def _tc_edgew(ea_ref, w1_ref, b1_ref, w2_ref, b2_ref, o_ref):
    h = jnp.dot(ea_ref[...], w1_ref[...], preferred_element_type=_f32)
    h = jnp.maximum(h + b1_ref[...], 0.0)
    z = jnp.dot(h, w2_ref[...], preferred_element_type=_f32) + b2_ref[...]
    o_ref[...] = jax.nn.sigmoid(z)


def _tc_dinv(deg_ref, o_ref):
    o_ref[...] = lax.rsqrt(deg_ref[...] + 1.0)


def _make_tc_pre(layer, with_bn):
    def body(x_ref, mom_ref, w_ref, dinv_ref, g_ref):
        x = x_ref[...]
        if with_bn:
            mom = mom_ref[...]
            mu = jnp.sum(mom[:, 0, :], axis=0, keepdims=True) / N
            msq = jnp.sum(mom[:, 1, :], axis=0, keepdims=True) / N
            inv = lax.rsqrt(msq - mu * mu + 1e-5)
            x = jnp.maximum((x - mu) * inv, 0.0)
        hw = jnp.dot(x, w_ref[...], preferred_element_type=_f32)
        g_ref[...] = hw * dinv_ref[:, layer:layer + 1]
    return body


def _make_tc_post(layer):
    def body(s_ref, g_ref, dinv_ref, b_ref, o_ref, mom_ref):
        bidx = pl.program_id(0)
        out = dinv_ref[:, layer:layer + 1] * (s_ref[...] + g_ref[...]) + b_ref[...]
        o_ref[...] = out
        rowid = bidx * RB + lax.broadcasted_iota(_i32, (RB, 1), 0)
        m = (rowid < N).astype(_f32)
        om = out * m
        mom_ref[0, 0, :] = jnp.sum(om, axis=0)
        mom_ref[0, 1, :] = jnp.sum(om * out, axis=0)
    return body


def _tc_post3(s_ref, g_ref, dinv_ref, b_ref, h0_ref, o_ref):
    out = dinv_ref[:, 3:4] * (s_ref[...] + g_ref[...]) + b_ref[...]
    o_ref[...] = jnp.maximum(out + h0_ref[...], 0.0)


def _tc_final(pp_ref, w_ref, b_ref, o_ref):
    pooled = jnp.max(pp_ref[...], axis=0)
    o_ref[...] = jnp.dot(pooled, w_ref[...], preferred_element_type=_f32) + b_ref[...]


def _rowspec(width):
    return pl.BlockSpec((RB, width), lambda b: (b, 0))


def _fullspec(shape):
    nd = len(shape)
    return pl.BlockSpec(shape, lambda b: (0,) * nd)


def kernel(x, edge_index, batch, dropout, edge_attr, device, params):
    row, col = edge_index[0], edge_index[1]

    perm = jnp.argsort(col)
    col_s = jnp.sort(col)
    row_s = row[perm]
    eb = jnp.searchsorted(col_s, jnp.arange(NW + 1, dtype=_i32) * CPW).astype(_i32)
    ebounds = jnp.zeros((NW, LN), _i32)
    ebounds = ebounds.at[:, 0].set(eb[:-1]).at[:, 1].set(eb[1:])
    col_sp = jnp.pad(col_s, (0, EPAD - E), constant_values=jnp.int32(2**30))
    row_sp = jnp.pad(row_s, (0, EPAD - E))
    batch_p = jnp.pad(batch, (0, NPAD - N))
    x_p = jnp.pad(x, ((0, NPAD - N), (0, 0)))
    ea_p = jnp.pad(edge_attr[perm], ((0, EPAD - E), (0, 0)))

    convs = [params["conv1"]] + list(params["hidden"])
    w1cat = jnp.concatenate([c["mlp_W1"] for c in convs], axis=1)
    b1cat = jnp.concatenate([c["mlp_b1"] for c in convs])[None, :]
    w2bd = jnp.zeros((NLAYER * ED, WL), _f32)
    b2v = jnp.zeros((1, WL), _f32)
    for l, c in enumerate(convs):
        w2bd = w2bd.at[l * ED:(l + 1) * ED, l].set(c["mlp_W2"][:, 0])
        b2v = b2v.at[0, l].set(c["mlp_b2"][0])

    w_sorted = pl.pallas_call(
        _tc_edgew,
        grid=(EPAD // RB,),
        in_specs=[_rowspec(ED), _fullspec((ED, NLAYER * ED)),
                  _fullspec((1, NLAYER * ED)), _fullspec((NLAYER * ED, WL)),
                  _fullspec((1, WL))],
        out_specs=_rowspec(WL),
        out_shape=jax.ShapeDtypeStruct((EPAD, WL), _f32),
    )(ea_p, w1cat, b1cat, w2bd, b2v)

    deg = _sc_deg(w_sorted, col_sp, ebounds)
    w_ls = [w_sorted[:, l] for l in range(NLAYER)]

    dinv = pl.pallas_call(
        _tc_dinv,
        grid=(NBLK,),
        in_specs=[_rowspec(WL)],
        out_specs=_rowspec(WL),
        out_shape=jax.ShapeDtypeStruct((NPAD, WL), _f32),
    )(deg)

    h = x_p
    mom = None
    h0 = None
    for l in range(NLAYER):
        with_bn = l > 0
        in_arrs = [h, mom if with_bn else jnp.zeros((NBLK, 2, C), _f32),
                   convs[l]["lin_W"], dinv]
        g = pl.pallas_call(
            _make_tc_pre(l, with_bn),
            grid=(NBLK,),
            in_specs=[_rowspec(C), _fullspec((NBLK, 2, C)),
                      _fullspec((C, C)), _rowspec(WL)],
            out_specs=_rowspec(C),
            out_shape=jax.ShapeDtypeStruct((NPAD, C), _f32),
        )(*in_arrs)

        s = _sc_layer(g, row_sp, col_sp, w_ls[l], ebounds)

        bias = convs[l]["bias"][None, :]
        if l < NLAYER - 1:
            h, mom = pl.pallas_call(
                _make_tc_post(l),
                grid=(NBLK,),
                in_specs=[_rowspec(C), _rowspec(C), _rowspec(WL),
                          _fullspec((1, C))],
                out_specs=[_rowspec(C), pl.BlockSpec((1, 2, C), lambda b: (b, 0, 0))],
                out_shape=[jax.ShapeDtypeStruct((NPAD, C), _f32),
                           jax.ShapeDtypeStruct((NBLK, 2, C), _f32)],
            )(s, g, dinv, bias)
            if l == 0:
                h0 = h
        else:
            h1 = pl.pallas_call(
                _tc_post3,
                grid=(NBLK,),
                in_specs=[_rowspec(C), _rowspec(C), _rowspec(WL),
                          _fullspec((1, C)), _rowspec(C)],
                out_specs=_rowspec(C),
                out_shape=jax.ShapeDtypeStruct((NPAD, C), _f32),
            )(s, g, dinv, bias, h0)

    pooled_part = _sc_segmax(h1, batch_p)

    linw = jnp.pad(params["lin_W"], ((0, 0), (0, C - NCLASS)))
    linb = jnp.pad(params["lin_b"], (0, C - NCLASS))[None, :]
    logits = pl.pallas_call(
        _tc_final,
        grid=(1,),
        in_specs=[_fullspec((NW, NGRAPH, C)), _fullspec((C, C)),
                  _fullspec((1, C))],
        out_specs=_fullspec((NGRAPH, C)),
        out_shape=jax.ShapeDtypeStruct((NGRAPH, C), _f32),
    )(pooled_part, linw, linb)
    return logits[:, :NCLASS]
